# Initial kernel scaffold; baseline (speedup 1.0000x reference)
import jax
import jax.numpy as jnp
from jax.experimental import pallas as pl


def kernel(seq_len, relative_bias):
    raise NotImplementedError("write your pallas kernel here")


# speedup vs baseline: 48.1579x; 48.1579x over previous
import jax
import jax.numpy as jnp
from jax import lax
from jax.experimental import pallas as pl
from jax.experimental.pallas import tpu as pltpu
from jax.experimental.pallas import tpu_sc as plsc

N_HEADS = 12
MAX_DIST = 128
L = 2 * MAX_DIST + 1
S = 2048
N_WORKERS = 32
ROWS_PER_WORKER = S // N_WORKERS
WIN = 2240
WIN0 = WIN + 16


def _sc_bias_kernel(table_hbm, out_hbm, table_v, w0, w1, w2, w3, w4, w5, w6, w7, sem):
    wins = (w0, w1, w2, w3, w4, w5, w6, w7)
    cid = lax.axis_index("c")
    sid = lax.axis_index("s")
    wid = sid * 2 + cid

    pltpu.sync_copy(table_hbm, table_v)

    vbase = 1976 - ROWS_PER_WORKER * wid

    lanes0 = lax.iota(jnp.int32, 16)

    def head_body(h, carry):
        tbase = h * L

        def build_chunk(k, c):
            a = vbase + k * 16 + lanes0
            idx = jnp.clip(a - (S - 1), -MAX_DIST, MAX_DIST) + MAX_DIST + tbase
            w0[pl.ds(k * 16, 16)] = plsc.load_gather(table_v, [idx])
            return c

        lax.fori_loop(0, WIN0 // 16, build_chunk, 0)

        def shift_chunk(k, c):
            m = k * 16
            for p in range(1, 8):
                wins[p][pl.ds(m, 16)] = plsc.load_gather(w0, [m + p + lanes0])
            return c

        lax.fori_loop(0, WIN // 16, shift_chunk, 0)

        row0 = h * S + ROWS_PER_WORKER * wid
        for t in range(ROWS_PER_WORKER):
            off = 71 - t
            p = off & 7
            m0 = off - p
            pltpu.async_copy(
                wins[p].at[pl.ds(m0, S)], out_hbm.at[row0 + t], sem
            )

        def drain(t, c):
            pltpu.make_async_copy(
                w0.at[pl.ds(0, S)], out_hbm.at[0], sem
            ).wait()
            return c

        lax.fori_loop(0, ROWS_PER_WORKER, drain, 0)
        return carry

    lax.fori_loop(0, N_HEADS, head_body, 0)


@jax.jit
def _run(table_flat):
    mesh = plsc.VectorSubcoreMesh(core_axis_name="c", subcore_axis_name="s")
    out = pl.kernel(
        _sc_bias_kernel,
        out_type=jax.ShapeDtypeStruct((N_HEADS * S, S), jnp.float32),
        mesh=mesh,
        compiler_params=pltpu.CompilerParams(
            needs_layout_passes=False, use_tc_tiling_on_sc=False
        ),
        scratch_types=[
            pltpu.VMEM((N_HEADS * L,), jnp.float32),
            pltpu.VMEM((WIN0,), jnp.float32),
            pltpu.VMEM((WIN,), jnp.float32),
            pltpu.VMEM((WIN,), jnp.float32),
            pltpu.VMEM((WIN,), jnp.float32),
            pltpu.VMEM((WIN,), jnp.float32),
            pltpu.VMEM((WIN,), jnp.float32),
            pltpu.VMEM((WIN,), jnp.float32),
            pltpu.VMEM((WIN,), jnp.float32),
            pltpu.SemaphoreType.DMA,
        ],
    )(table_flat)
    return out.reshape(N_HEADS, S, S)


def kernel(seq_len, relative_bias):
    del seq_len
    return _run(relative_bias.reshape(-1))


# speedup vs baseline: 56.4704x; 1.1726x over previous
import jax
import jax.numpy as jnp
from jax import lax
from jax.experimental import pallas as pl
from jax.experimental.pallas import tpu as pltpu
from jax.experimental.pallas import tpu_sc as plsc

N_HEADS = 12
MAX_DIST = 128
L = 2 * MAX_DIST + 1
S = 2048
N_WORKERS = 32
ROWS_PER_WORKER = S // N_WORKERS
WIN = 2256


def _sc_bias_kernel(
    table_hbm, out_hbm, table_v,
    a0, a1, a2, a3, a4, a5, a6, a7,
    b0, b1, b2, b3, b4, b5, b6, b7,
    sem_a, sem_b,
):
    wins_a = (a0, a1, a2, a3, a4, a5, a6, a7)
    wins_b = (b0, b1, b2, b3, b4, b5, b6, b7)

    cid = lax.axis_index("c")
    sid = lax.axis_index("s")
    wid = sid * 2 + cid

    pltpu.sync_copy(table_hbm, table_v)

    vbase = 1976 - ROWS_PER_WORKER * wid

    lanes0 = lax.iota(jnp.int32, 16)

    def clip_idx(x, tbase):
        return jnp.clip(x - (S - 1), -MAX_DIST, MAX_DIST) + MAX_DIST + tbase

    def build(h, wins):
        tbase = h * L
        for p in range(1, 8):
            idx0 = clip_idx(vbase + p + lanes0, tbase)
            wins[p][pl.ds(0, 16)] = plsc.load_gather(table_v, [idx0])

        def chunk(k, c):
            m = k * 16
            vals = plsc.load_gather(table_v, [clip_idx(vbase + m + lanes0, tbase)])
            wins[0][pl.ds(m, 16)] = vals
            for p in range(1, 8):
                wins[p][pl.ds(m - p, 16)] = vals
            return c

        lax.fori_loop(1, WIN // 16, chunk, 0)
        idxz = clip_idx(vbase + lanes0, tbase)
        wins[0][pl.ds(0, 16)] = plsc.load_gather(table_v, [idxz])

    def issue(h, wins, sem):
        row0 = h * S + ROWS_PER_WORKER * wid
        for t in range(ROWS_PER_WORKER):
            off = 71 - t
            p = off & 7
            m0 = off - p
            pltpu.async_copy(wins[p].at[pl.ds(m0, S)], out_hbm.at[row0 + t], sem)

    def drain(sem):
        def one(t, c):
            pltpu.make_async_copy(a0.at[pl.ds(0, S)], out_hbm.at[0], sem).wait()
            return c

        lax.fori_loop(0, ROWS_PER_WORKER, one, 0)

    build(0, wins_a)

    def body(g, c):
        h_a = 2 * g
        h_b = 2 * g + 1
        issue(h_a, wins_a, sem_a)
        build(h_b, wins_b)
        drain(sem_a)
        issue(h_b, wins_b, sem_b)
        build(jnp.minimum(h_a + 2, N_HEADS - 1), wins_a)
        drain(sem_b)
        return c

    lax.fori_loop(0, N_HEADS // 2, body, 0)


@jax.jit
def _run(table_flat):
    mesh = plsc.VectorSubcoreMesh(core_axis_name="c", subcore_axis_name="s")
    win_t = pltpu.VMEM((WIN,), jnp.float32)
    out = pl.kernel(
        _sc_bias_kernel,
        out_type=jax.ShapeDtypeStruct((N_HEADS * S, S), jnp.float32),
        mesh=mesh,
        compiler_params=pltpu.CompilerParams(
            needs_layout_passes=False, use_tc_tiling_on_sc=False
        ),
        scratch_types=[pltpu.VMEM((N_HEADS * L,), jnp.float32)]
        + [win_t] * 16
        + [pltpu.SemaphoreType.DMA, pltpu.SemaphoreType.DMA],
    )(table_flat)
    return out.reshape(N_HEADS, S, S)


def kernel(seq_len, relative_bias):
    del seq_len
    return _run(relative_bias.reshape(-1))
